# Initial kernel scaffold; baseline (speedup 1.0000x reference)
import jax
import jax.numpy as jnp
from jax.experimental import pallas as pl


def kernel(s, W, b):
    raise NotImplementedError("write your pallas kernel here")


# speedup vs baseline: 18.4319x; 18.4319x over previous
import functools

import jax
import jax.numpy as jnp
from jax.experimental import pallas as pl
from jax.experimental.pallas import tpu as pltpu

_K = 40


def _body(s_ref, w_ref, b_ref, out_ref, v_ref):
    logits = jax.lax.dot_general(
        s_ref[...], w_ref[...],
        dimension_numbers=(((1,), (1,)), ((), ())),
        preferred_element_type=jnp.float32,
    )
    logits = logits + b_ref[...]
    bits = jax.lax.bitcast_convert_type(logits, jnp.int32)
    v = bits ^ jnp.bitwise_and(jax.lax.shift_right_arithmetic(bits, 31),
                               jnp.int32(0x7FFFFFFF))
    v_ref[...] = v

    rb = v.shape[0]
    init = jnp.full((rb, 1), jnp.iinfo(jnp.int32).min, dtype=jnp.int32)

    def step(i, prefix):
        bit = jax.lax.shift_left(jnp.int32(1), jnp.int32(31) - i)
        cand = prefix + bit
        cnt = jnp.sum((v_ref[...] >= cand).astype(jnp.int32), axis=1,
                      keepdims=True)
        return jnp.where(cnt >= _K, cand, prefix)

    thresh = jax.lax.fori_loop(0, 32, step, init)
    out_ref[...] = jnp.where(v_ref[...] >= thresh, jnp.float32(1.0),
                             jnp.float32(0.0))


@functools.partial(jax.jit, static_argnames=())
def kernel(s, W, b):
    B, D = s.shape
    N = W.shape[0]
    Rb = 256
    b2 = b.reshape(1, N)
    grid = (B // Rb,)
    return pl.pallas_call(
        _body,
        grid=grid,
        in_specs=[
            pl.BlockSpec((Rb, D), lambda i: (i, 0)),
            pl.BlockSpec((N, D), lambda i: (0, 0)),
            pl.BlockSpec((1, N), lambda i: (0, 0)),
        ],
        out_specs=pl.BlockSpec((Rb, N), lambda i: (i, 0)),
        out_shape=jax.ShapeDtypeStruct((B, N), jnp.float32),
        scratch_shapes=[pltpu.VMEM((Rb, N), jnp.int32)],
        compiler_params=pltpu.CompilerParams(
            dimension_semantics=("arbitrary",),
        ),
    )(s, W, b2)


# speedup vs baseline: 22.0423x; 1.1959x over previous
import functools

import jax
import jax.numpy as jnp
from jax.experimental import pallas as pl
from jax.experimental.pallas import tpu as pltpu

_K = 40


def _body(s_ref, w_ref, b_ref, out_ref, v_ref):
    logits = jax.lax.dot_general(
        s_ref[...], w_ref[...],
        dimension_numbers=(((1,), (1,)), ((), ())),
        preferred_element_type=jnp.float32,
    )
    logits = logits + b_ref[...]
    bits = jax.lax.bitcast_convert_type(logits, jnp.int32)
    v = bits ^ jnp.bitwise_and(jax.lax.shift_right_arithmetic(bits, 31),
                               jnp.int32(0x7FFFFFFF))
    v_ref[...] = v

    rb = v.shape[0]
    init_prefix = jnp.full((rb, 1), jnp.iinfo(jnp.int32).min, dtype=jnp.int32)
    init_cnt = jnp.full((rb, 1), v.shape[1], dtype=jnp.int32)

    def cond(carry):
        i, _, cnt = carry
        return jnp.logical_and(i < 32, jnp.any(cnt != _K))

    def step(carry):
        i, prefix, cnt = carry
        bit = jax.lax.shift_left(jnp.int32(1), jnp.int32(31) - i)
        cand = prefix + bit
        cntc = jnp.sum((v_ref[...] >= cand).astype(jnp.int32), axis=1,
                       keepdims=True)
        accept = cntc >= _K
        return (i + jnp.int32(1),
                jnp.where(accept, cand, prefix),
                jnp.where(accept, cntc, cnt))

    _, thresh, _ = jax.lax.while_loop(
        cond, step, (jnp.int32(0), init_prefix, init_cnt))
    out_ref[...] = jnp.where(v_ref[...] >= thresh, jnp.float32(1.0),
                             jnp.float32(0.0))


@functools.partial(jax.jit, static_argnames=())
def kernel(s, W, b):
    B, D = s.shape
    N = W.shape[0]
    Rb = 256
    b2 = b.reshape(1, N)
    grid = (B // Rb,)
    return pl.pallas_call(
        _body,
        grid=grid,
        in_specs=[
            pl.BlockSpec((Rb, D), lambda i: (i, 0)),
            pl.BlockSpec((N, D), lambda i: (0, 0)),
            pl.BlockSpec((1, N), lambda i: (0, 0)),
        ],
        out_specs=pl.BlockSpec((Rb, N), lambda i: (i, 0)),
        out_shape=jax.ShapeDtypeStruct((B, N), jnp.float32),
        scratch_shapes=[pltpu.VMEM((Rb, N), jnp.int32)],
        compiler_params=pltpu.CompilerParams(
            dimension_semantics=("parallel",),
        ),
    )(s, W, b2)
